# baseline (speedup 1.0000x reference)
import functools

import jax
import jax.numpy as jnp
from jax import lax
from jax.experimental import pallas as pl
from jax.experimental.pallas import tpu as pltpu
from jax.experimental.pallas import tpu_sc as plsc

N = 50000
D = 64
H = 32
E = 800000
B = 4096
NC, NS, L = 2, 16, 16
G = 256
EPW = 51200
NG = EPW // G
NROW = NS * NG
DPT = B // NS
ZR = 125
NPT = N // NS


def _sc_body(emb2, edges, drugs,
             gamma_out,
             xflat,
             dacc,
             acc_sp,
             ebuf,
             gidx,
             rows,
             gam_v,
             esem, gsem, ssem):
    c = lax.axis_index("c")
    s = lax.axis_index("s")
    zvec = jnp.zeros((L,), jnp.float32)
    half = c * N
    nbase = s * NPT
    rbase = s * NG

    for z in range(NPT // ZR):
        pltpu.sync_copy(
            emb2.at[pl.ds(half + nbase + z * ZR, ZR)],
            rows.at[0, pl.ds(0, ZR)])
        pltpu.sync_copy(
            rows.at[0, pl.ds(0, ZR)],
            xflat.at[pl.ds(half + nbase + z * ZR, ZR)])
    plsc.subcore_barrier()

    def _gather_drugs(base, hh):
        pltpu.sync_copy(drugs.at[pl.ds(s * DPT + hh * 128, 128)],
                        gidx.at[0, 0])
        for j in range(128 // L):
            gidx[0, 0, pl.ds(j * L, L)] = (
                gidx[0, 0, pl.ds(j * L, L)] + base)
        pltpu.async_copy(
            xflat.at[gidx.at[0, 0]], rows.at[0, pl.ds(0, 128)],
            gsem.at[0]).wait()

    for hh in range(DPT // 128):
        _gather_drugs(half, hh)
        pltpu.sync_copy(rows.at[0, pl.ds(0, 128)],
                        dacc.at[c, pl.ds(s * DPT + hh * 128, 128)])

    def _zero_acc():
        def _zfill(i, cc):
            rows[0, i, 0:L] = zvec
            rows[0, i, L:H] = zvec
            return cc
        lax.fori_loop(0, ZR, _zfill, 0)
        zcps = [pltpu.async_copy(
            rows.at[0, pl.ds(0, ZR)],
            acc_sp.at[pl.ds(nbase + z * ZR, ZR)], gsem.at[1])
            for z in range(NPT // ZR)]
        for zd in zcps:
            zd.wait()

    _zero_acc()
    plsc.subcore_barrier()

    def _fetch(g):
        row = lax.min(rbase + g, NROW - 1)
        slot = lax.rem(g, 5)
        return pltpu.async_copy(
            edges.at[row], ebuf.at[slot], esem.at[slot])

    def _prop(k, carry):
        rd_base = ((k - 1) & 1) * (2 * N) + half
        wr_base = (k & 1) * (2 * N) + half

        _fetch(0)
        _fetch(1)

        def _body(g, cc):
            sl = lax.rem(g, 3)
            e5 = lax.rem(g, 5)

            @pl.when(jnp.logical_and(g >= 3, g < NG))
            def _():
                for r in range(2):
                    pltpu.make_async_copy(
                        rows.at[sl, pl.ds(r * 128, 128)],
                        acc_sp.at[ebuf.at[e5, 1, r]], ssem.at[sl]).wait()

            @pl.when(g < NG)
            def _():
                _fetch(g + 2)
                pltpu.make_async_copy(
                    edges.at[rbase], ebuf.at[e5], esem.at[e5]).wait()
                for r in range(2):
                    for j in range(128 // L):
                        gidx[sl, r, pl.ds(j * L, L)] = (
                            ebuf[e5, 0, r, pl.ds(j * L, L)] + rd_base)
                for r in range(2):
                    pltpu.async_copy(
                        xflat.at[gidx.at[sl, r]],
                        rows.at[sl, pl.ds(r * 128, 128)], gsem.at[sl])

            @pl.when(g >= 2)
            def _():
                h = g - 2
                slh = lax.rem(h, 3)
                eh = lax.rem(h, 5)
                for r in range(2):
                    pltpu.make_async_copy(
                        xflat.at[gidx.at[slh, r]],
                        rows.at[slh, pl.ds(r * 128, 128)],
                        gsem.at[slh]).wait()

                def _scale(gg, cc2):
                    vr = lax.div(gg, 8)
                    vo = lax.rem(gg, 8) * L
                    vv = plsc.bitcast(
                        ebuf[eh, 2, vr, pl.ds(vo, L)], jnp.float32)
                    e0 = gg * L
                    for t in range(L):
                        v = vv[t]
                        rows[slh, e0 + t, 0:L] = rows[slh, e0 + t, 0:L] * v
                        rows[slh, e0 + t, L:H] = rows[slh, e0 + t, L:H] * v
                    return cc2
                lax.fori_loop(0, G // L, _scale, 0)
                for r in range(2):
                    pltpu.async_copy(
                        rows.at[slh, pl.ds(r * 128, 128)],
                        acc_sp.at[ebuf.at[eh, 1, r]], ssem.at[slh],
                        add=True)
            return cc
        lax.fori_loop(0, NG + 2, _body, 0)
        for gg in (NG - 3, NG - 2, NG - 1):
            for r in range(2):
                pltpu.make_async_copy(
                    rows.at[gg % 3, pl.ds(r * 128, 128)],
                    acc_sp.at[ebuf.at[gg % 5, 1, r]],
                    ssem.at[gg % 3]).wait()
        for gg in (NG, NG + 1):
            pltpu.make_async_copy(
                edges.at[rbase], ebuf.at[gg % 5], esem.at[gg % 5]).wait()
        plsc.subcore_barrier()

        pltpu.sync_copy(
            acc_sp.at[pl.ds(nbase, NPT)],
            xflat.at[pl.ds(wr_base + nbase, NPT)])
        plsc.subcore_barrier()

        for hh in range(DPT // 128):
            _gather_drugs(wr_base, hh)
            pltpu.sync_copy(dacc.at[c, pl.ds(s * DPT + hh * 128, 128)],
                            rows.at[1, pl.ds(0, 128)])

            def _accm(e, cc):
                rows[1, e, 0:L] = rows[1, e, 0:L] + rows[0, e, 0:L]
                rows[1, e, L:H] = rows[1, e, L:H] + rows[0, e, L:H]
                return cc
            lax.fori_loop(0, 128, _accm, 0)
            pltpu.sync_copy(rows.at[1, pl.ds(0, 128)],
                            dacc.at[c, pl.ds(s * DPT + hh * 128, 128)])
        _zero_acc()
        plsc.subcore_barrier()
        return carry
    lax.fori_loop(1, 5, _prop, 0)

    pltpu.sync_copy(dacc.at[c, pl.ds(s * DPT, DPT)],
                    rows.at[0, pl.ds(0, DPT)])

    def _gam16(g, carry):
        rows_idx = g * L + lax.iota(jnp.int32, L)

        def _dim(d, ss):
            col = plsc.load_gather(
                rows.at[0], [rows_idx, jnp.full((L,), d, jnp.int32)])
            return ss + col * col
        ss = lax.fori_loop(0, H, _dim, jnp.zeros((L,), jnp.float32))
        gam_v[pl.ds(g * L, L)] = ss * 0.04
        return carry
    lax.fori_loop(0, DPT // L, _gam16, 0)
    pltpu.sync_copy(gam_v, gamma_out.at[c, pl.ds(s * DPT, DPT)])


@jax.jit
def _run(emb2, edges, drugs):
    mesh = plsc.VectorSubcoreMesh(core_axis_name="c", subcore_axis_name="s")
    f = pl.kernel(
        _sc_body,
        out_type=jax.ShapeDtypeStruct((NC, B), jnp.float32),
        mesh=mesh,
        compiler_params=pltpu.CompilerParams(
            needs_layout_passes=False, use_tc_tiling_on_sc=False),
        scratch_types=[
            pltpu.HBM((2 * NC * N, H), jnp.float32),
            pltpu.HBM((NC, B, H), jnp.float32),
            pltpu.VMEM_SHARED((N, H), jnp.float32),
            pltpu.VMEM((5, 3, 2, 128), jnp.int32),
            pltpu.VMEM((3, 2, 128), jnp.int32),
            pltpu.VMEM((3, G, H), jnp.float32),
            pltpu.VMEM((DPT,), jnp.float32),
            pltpu.SemaphoreType.DMA((5,)),
            pltpu.SemaphoreType.DMA((3,)),
            pltpu.SemaphoreType.DMA((3,)),
        ],
    )
    return f(emb2, edges, drugs)


def kernel(emb_weight, edge_vals, edge_index, drugs):
    emb2 = (emb_weight.reshape(N, NC, H)
            .transpose(1, 0, 2)
            .reshape(NC * N, H))
    pad = NS * EPW - E
    srcs = jnp.concatenate(
        [edge_index[1], jnp.zeros((pad,), jnp.int32)]).reshape(-1, 2, 128)
    dsts = jnp.concatenate(
        [edge_index[0], jnp.zeros((pad,), jnp.int32)]).reshape(-1, 2, 128)
    vals = lax.bitcast_convert_type(
        jnp.concatenate([edge_vals, jnp.zeros((pad,), jnp.float32)]),
        jnp.int32).reshape(-1, 2, 128)
    edges = jnp.stack([srcs, dsts, vals], axis=1)
    parts = _run(emb2, edges, drugs)
    return parts[0] + parts[1]


# speedup vs baseline: 1.1024x; 1.1024x over previous
import functools

import jax
import jax.numpy as jnp
from jax import lax
from jax.experimental import pallas as pl
from jax.experimental.pallas import tpu as pltpu
from jax.experimental.pallas import tpu_sc as plsc

N = 50000
D = 64
H = 32
E = 800000
B = 4096
NC, NS, L = 2, 16, 16
EPW = 51200
RPT = EPW // 128
NQ = RPT // 4
NROW = NS * RPT
DPT = B // NS
ZR = 125
NPT = N // NS


def _sc_body(emb2, edges, drugs,
             gamma_out,
             xflat,
             acc_sp,
             ebuf,
             gidx,
             rows,
             acc_v, gam_v,
             esem,
             gsem0, gsem1, gsem2, gsem3,
             ssem0, ssem1, ssem2, ssem3):
    c = lax.axis_index("c")
    s = lax.axis_index("s")
    gsem = (gsem0, gsem1, gsem2, gsem3)
    ssem = (ssem0, ssem1, ssem2, ssem3)
    zvec = jnp.zeros((L,), jnp.float32)
    half = c * N
    nbase = s * NPT
    rbase = s * RPT

    for z in range(NPT // ZR):
        pltpu.sync_copy(
            emb2.at[pl.ds(half + nbase + z * ZR, ZR)],
            rows.at[0, pl.ds(0, ZR)])
        pltpu.sync_copy(
            rows.at[0, pl.ds(0, ZR)],
            xflat.at[pl.ds(half + nbase + z * ZR, ZR)])
    plsc.subcore_barrier()

    def _drug_rows(base, fn):
        for hh in range(DPT // 128):
            pltpu.sync_copy(drugs.at[pl.ds(s * DPT + hh * 128, 128)],
                            gidx.at[0])
            for j in range(128 // L):
                gidx[0, pl.ds(j * L, L)] = gidx[0, pl.ds(j * L, L)] + base
            pltpu.async_copy(xflat.at[gidx.at[0]], rows.at[0], gsem0).wait()

            def _upd(e, carry):
                fn(hh * 128 + e, e)
                return carry
            lax.fori_loop(0, 128, _upd, 0)

    def _seed(a, e):
        acc_v[a, 0:L] = rows[0, e, 0:L]
        acc_v[a, L:H] = rows[0, e, L:H]
    _drug_rows(half, _seed)

    def _fetch(q):
        row = lax.min(rbase + q * 4, NROW - 4)
        slot = lax.rem(q, 3)
        return pltpu.async_copy(
            edges.at[pl.ds(row, 4)], ebuf.at[slot], esem.at[slot])

    def _zero_acc(src_slot, sem):
        def _zfill(i, cc):
            rows[src_slot, i, 0:L] = zvec
            rows[src_slot, i, L:H] = zvec
            return cc
        lax.fori_loop(0, ZR, _zfill, 0)
        return [pltpu.async_copy(
            rows.at[src_slot, pl.ds(0, ZR)],
            acc_sp.at[pl.ds(nbase + z * ZR, ZR)], sem)
            for z in range(NPT // ZR)]

    for zd in _zero_acc(0, gsem1):
        zd.wait()
    plsc.subcore_barrier()

    def _prop(k, carry):
        rd_base = ((k - 1) & 1) * (2 * N) + half
        wr_base = (k & 1) * (2 * N) + half

        _fetch(0)

        def _quad_body(q, drain_prev):
            p = lax.rem(q, 3)
            _fetch(q + 1)
            pltpu.make_async_copy(
                edges.at[pl.ds(rbase, 4)], ebuf.at[p], esem.at[p]).wait()

            gd = []
            for b in range(4):
                if drain_prev:
                    pltpu.make_async_copy(
                        rows.at[b], acc_sp.at[ebuf.at[p, b, 1]],
                        ssem[b]).wait()
                for j in range(128 // L):
                    gidx[b, pl.ds(j * L, L)] = (
                        ebuf[p, b, 0, pl.ds(j * L, L)] + rd_base)
                gd.append(pltpu.async_copy(
                    xflat.at[gidx.at[b]], rows.at[b], gsem[b]))

            for b in range(4):
                gd[b].wait()

                def _scale(g, cc2):
                    e0 = g * L
                    vv = plsc.bitcast(
                        ebuf[p, b, 2, pl.ds(e0, L)], jnp.float32)
                    for t in range(L):
                        v = vv[t]
                        rows[b, e0 + t, 0:L] = rows[b, e0 + t, 0:L] * v
                        rows[b, e0 + t, L:H] = rows[b, e0 + t, L:H] * v
                    return cc2
                lax.fori_loop(0, 128 // L, _scale, 0)
                pltpu.async_copy(
                    rows.at[b], acc_sp.at[ebuf.at[p, b, 1]], ssem[b],
                    add=True)

        _quad_body(0, False)

        def _quad(q, cc):
            _quad_body(q, True)
            return cc
        lax.fori_loop(1, NQ, _quad, 0)
        for b in range(4):
            pltpu.make_async_copy(
                rows.at[b], acc_sp.at[ebuf.at[(NQ - 1) % 3, b, 1]],
                ssem[b]).wait()
        pltpu.make_async_copy(
            edges.at[pl.ds(rbase, 4)], ebuf.at[NQ % 3],
            esem.at[NQ % 3]).wait()
        plsc.subcore_barrier()

        pltpu.sync_copy(
            acc_sp.at[pl.ds(nbase, NPT)],
            xflat.at[pl.ds(wr_base + nbase, NPT)])
        plsc.subcore_barrier()

        zcps = _zero_acc(3, gsem1)

        def _accm(a, e):
            acc_v[a, 0:L] = acc_v[a, 0:L] + rows[0, e, 0:L]
            acc_v[a, L:H] = acc_v[a, L:H] + rows[0, e, L:H]
        _drug_rows(wr_base, _accm)
        for zd in zcps:
            zd.wait()
        plsc.subcore_barrier()
        return carry
    lax.fori_loop(1, 5, _prop, 0)

    def _gam16(g, carry):
        rows_idx = g * L + lax.iota(jnp.int32, L)

        def _dim(d, ss):
            col = plsc.load_gather(
                acc_v, [rows_idx, jnp.full((L,), d, jnp.int32)])
            return ss + col * col
        ss = lax.fori_loop(0, H, _dim, jnp.zeros((L,), jnp.float32))
        gam_v[pl.ds(g * L, L)] = ss * 0.04
        return carry
    lax.fori_loop(0, DPT // L, _gam16, 0)
    pltpu.sync_copy(gam_v, gamma_out.at[c, pl.ds(s * DPT, DPT)])


@jax.jit
def _run(emb2, edges, drugs):
    mesh = plsc.VectorSubcoreMesh(core_axis_name="c", subcore_axis_name="s")
    f = pl.kernel(
        _sc_body,
        out_type=jax.ShapeDtypeStruct((NC, B), jnp.float32),
        mesh=mesh,
        compiler_params=pltpu.CompilerParams(
            needs_layout_passes=False, use_tc_tiling_on_sc=False),
        scratch_types=[
            pltpu.HBM((2 * NC * N, H), jnp.float32),
            pltpu.VMEM_SHARED((N, H), jnp.float32),
            pltpu.VMEM((3, 4, 3, 128), jnp.int32),
            pltpu.VMEM((4, 128), jnp.int32),
            pltpu.VMEM((4, 128, H), jnp.float32),
            pltpu.VMEM((DPT, H), jnp.float32),
            pltpu.VMEM((DPT,), jnp.float32),
            pltpu.SemaphoreType.DMA((3,)),
            pltpu.SemaphoreType.DMA,
            pltpu.SemaphoreType.DMA,
            pltpu.SemaphoreType.DMA,
            pltpu.SemaphoreType.DMA,
            pltpu.SemaphoreType.DMA,
            pltpu.SemaphoreType.DMA,
            pltpu.SemaphoreType.DMA,
            pltpu.SemaphoreType.DMA,
        ],
    )
    return f(emb2, edges, drugs)


def kernel(emb_weight, edge_vals, edge_index, drugs):
    emb2 = (emb_weight.reshape(N, NC, H)
            .transpose(1, 0, 2)
            .reshape(NC * N, H))
    pad = NS * EPW - E
    srcs = jnp.concatenate(
        [edge_index[1], jnp.zeros((pad,), jnp.int32)]).reshape(-1, 128)
    dsts = jnp.concatenate(
        [edge_index[0], jnp.zeros((pad,), jnp.int32)]).reshape(-1, 128)
    vals = lax.bitcast_convert_type(
        jnp.concatenate([edge_vals, jnp.zeros((pad,), jnp.float32)]),
        jnp.int32).reshape(-1, 128)
    edges = jnp.stack([srcs, dsts, vals], axis=1)
    parts = _run(emb2, edges, drugs)
    return parts[0] + parts[1]
